# baseline (speedup 1.0000x reference)
import jax
import jax.numpy as jnp
from jax.experimental import pallas as pl
from jax.experimental.pallas import tpu as pltpu

_N = 1048576
_H = 64
_R = 8192
_NT = (_N * _H) // (_R * 128)


def _sum_kernel(mb_ref, out_ref):
    i = pl.program_id(0)
    part = jnp.sum(mb_ref[...], axis=0, keepdims=True)

    @pl.when(i == 0)
    def _init():
        out_ref[...] = part

    @pl.when(i != 0)
    def _acc():
        out_ref[...] = out_ref[...] + part


def kernel(query_embedding, embedding, timestamp, current_timestamp,
           memory_bank, timestamps):
    mbw = memory_bank.reshape((_N * _H) // 128, 128)
    s = pl.pallas_call(
        _sum_kernel,
        grid=(_NT,),
        in_specs=[pl.BlockSpec((_R, 128), lambda i: (i, 0))],
        out_specs=pl.BlockSpec((1, 128), lambda i: (0, 0)),
        out_shape=jax.ShapeDtypeStruct((1, 128), jnp.float32),
    )(mbw)
    return s[0, :64] + s[0, 64:] + 0.0 * embedding


# speedup vs baseline: 1.1752x; 1.1752x over previous
import functools

import jax
import jax.numpy as jnp
from jax import lax
from jax.experimental import pallas as pl
from jax.experimental.pallas import tpu as pltpu
from jax.experimental.pallas import tpu_sc as plsc

_N = 1048576
_H = 64
_TAU = 1000.0

_NW = 32
_RPW = _N // _NW
_CR = 256
_NCH = _RPW // _CR


def _sc_weighted_sum_body(mb_hbm, ts_hbm, out_hbm,
                          mb_v0, mb_v1, ts_v0, ts_v1, stage,
                          sem_mb0, sem_mb1, sem_ts0, sem_ts1):
    mb_bufs = (mb_v0, mb_v1)
    ts_bufs = (ts_v0, ts_v1)
    mb_sems = (sem_mb0, sem_mb1)
    ts_sems = (sem_ts0, sem_ts1)

    wid = lax.axis_index("s") * 2 + lax.axis_index("c")
    base = wid * _RPW

    def start(chunk, b):
        row0 = base + chunk * _CR
        pltpu.make_async_copy(mb_hbm.at[pl.ds(row0, _CR)], mb_bufs[b],
                              mb_sems[b]).start()
        pltpu.make_async_copy(ts_hbm.at[pl.ds(row0, _CR)], ts_bufs[b],
                              ts_sems[b]).start()

    start(0, 0)
    start(1, 1)

    inv_tau = 1.0 / _TAU

    def gbody(g, accs):
        for b in range(2):
            chunk = g * 2 + b
            pltpu.make_async_copy(mb_hbm.at[pl.ds(0, _CR)], mb_bufs[b],
                                  mb_sems[b]).wait()
            pltpu.make_async_copy(ts_hbm.at[pl.ds(0, _CR)], ts_bufs[b],
                                  ts_sems[b]).wait()

            def rbody(j16, accs, b=b):
                a0, a1, a2, a3 = accs
                r = mb_bufs[b]
                t = ts_bufs[b]
                wv = jnp.exp(t[pl.ds(j16 * 16, 16)] * inv_tau)
                for jj in range(16):
                    j = j16 * 16 + jj
                    w = wv[jj]
                    a0 = a0 + r[j, pl.ds(0, 16)] * w
                    a1 = a1 + r[j, pl.ds(16, 16)] * w
                    a2 = a2 + r[j, pl.ds(32, 16)] * w
                    a3 = a3 + r[j, pl.ds(48, 16)] * w
                return (a0, a1, a2, a3)

            accs = lax.fori_loop(0, _CR // 16, rbody, accs)

            @pl.when(chunk + 2 < _NCH)
            def _(chunk=chunk, b=b):
                start(chunk + 2, b)
        return accs

    zero = jnp.zeros((16,), jnp.float32)
    accs = lax.fori_loop(0, _NCH // 2, gbody, (zero, zero, zero, zero))

    for q in range(4):
        stage[pl.ds(q * 16, 16)] = accs[q]
    pltpu.sync_copy(stage, out_hbm.at[wid])


_sc_weighted_sum = functools.partial(
    pl.kernel,
    out_type=jax.ShapeDtypeStruct((_NW, _H), jnp.float32),
    mesh=plsc.VectorSubcoreMesh(core_axis_name="c", subcore_axis_name="s"),
    scratch_types=[
        pltpu.VMEM((_CR, _H), jnp.float32),
        pltpu.VMEM((_CR, _H), jnp.float32),
        pltpu.VMEM((_CR,), jnp.float32),
        pltpu.VMEM((_CR,), jnp.float32),
        pltpu.VMEM((_H,), jnp.float32),
        pltpu.SemaphoreType.DMA,
        pltpu.SemaphoreType.DMA,
        pltpu.SemaphoreType.DMA,
        pltpu.SemaphoreType.DMA,
    ],
)(_sc_weighted_sum_body)


def _argmin_gather_kernel(ts_ref, mb_hbm, min_ref, row_ref, sem):
    x = ts_ref[...]
    m = jnp.min(x)
    r, c = x.shape
    lin = (jax.lax.broadcasted_iota(jnp.int32, (r, c), 0) * c
           + jax.lax.broadcasted_iota(jnp.int32, (r, c), 1))
    cand = jnp.where(x == m, lin, jnp.int32(2147483647))
    idx = jnp.min(cand)
    min_ref[0] = m
    cp = pltpu.make_async_copy(mb_hbm.at[pl.ds(idx, 1)], row_ref, sem)
    cp.start()
    cp.wait()


def kernel(query_embedding, embedding, timestamp, current_timestamp,
           memory_bank, timestamps):
    partials = _sc_weighted_sum(memory_bank, timestamps)

    min_ts, row = pl.pallas_call(
        _argmin_gather_kernel,
        in_specs=[
            pl.BlockSpec(memory_space=pltpu.VMEM),
            pl.BlockSpec(memory_space=pl.ANY),
        ],
        out_specs=[
            pl.BlockSpec(memory_space=pltpu.SMEM),
            pl.BlockSpec(memory_space=pltpu.VMEM),
        ],
        out_shape=[
            jax.ShapeDtypeStruct((1,), jnp.float32),
            jax.ShapeDtypeStruct((1, _H), jnp.float32),
        ],
        scratch_shapes=[pltpu.SemaphoreType.DMA],
    )(timestamps.reshape(_N // 128, 128), memory_bank)

    s = jnp.sum(partials, axis=0)
    scale = jnp.exp(-current_timestamp / _TAU)
    w_old = jnp.exp((min_ts[0] - current_timestamp) / _TAU)
    w_new = jnp.exp((timestamp - current_timestamp) / _TAU)
    out = (s * scale - w_old * row[0] + w_new * embedding) * (1.0 / _N)
    return out
